# baseline (speedup 1.0000x reference)
import jax
import jax.numpy as jnp
from jax.experimental import pallas as pl


def _add_kernel(x_ref, w_ref, o_ref, *, t):
    w = w_ref[...]
    w_rep = jnp.concatenate([w] * t, axis=1)
    o_ref[...] = x_ref[...] + w_rep[None, :, :]


def kernel(x, W):
    batch, n, t, f = x.shape
    x2 = x.reshape(batch, n, t * f)
    rb = 2000
    import functools
    out = pl.pallas_call(
        functools.partial(_add_kernel, t=t),
        grid=(batch, n // rb),
        in_specs=[
            pl.BlockSpec((1, rb, t * f), lambda b, i: (b, i, 0)),
            pl.BlockSpec((rb, f), lambda b, i: (i, 0)),
        ],
        out_specs=pl.BlockSpec((1, rb, t * f), lambda b, i: (b, i, 0)),
        out_shape=jax.ShapeDtypeStruct(x2.shape, x.dtype),
    )(x2, W)
    return out.reshape(x.shape)


# speedup vs baseline: 3.8534x; 3.8534x over previous
import functools

import jax
import jax.numpy as jnp
from jax.experimental import pallas as pl


def _add_kernel(x_ref, w_ref, o_ref):
    o_ref[...] = x_ref[...] + w_ref[...][None, None, :, :]


def kernel(x, W):
    batch, n, t, f = x.shape
    xt = jnp.transpose(x, (0, 2, 1, 3))
    nb = 2000
    out_t = pl.pallas_call(
        _add_kernel,
        grid=(n // nb, batch, t),
        in_specs=[
            pl.BlockSpec((1, 1, nb, f), lambda i, b, s: (b, s, i, 0)),
            pl.BlockSpec((nb, f), lambda i, b, s: (i, 0)),
        ],
        out_specs=pl.BlockSpec((1, 1, nb, f), lambda i, b, s: (b, s, i, 0)),
        out_shape=jax.ShapeDtypeStruct((batch, t, n, f), x.dtype),
    )(xt, W)
    return jnp.transpose(out_t, (0, 2, 1, 3))


# speedup vs baseline: 6.1122x; 1.5862x over previous
import functools

import jax
import jax.numpy as jnp
from jax.experimental import pallas as pl


def _add_kernel(x_ref, w_ref, o_ref):
    o_ref[...] = x_ref[...] + w_ref[...][None, None, :, :]


def kernel(x, W):
    batch, n, t, f = x.shape
    xt = jnp.transpose(x, (0, 2, 1, 3))
    nb = 10000
    out_t = pl.pallas_call(
        _add_kernel,
        grid=(n // nb, batch, t),
        in_specs=[
            pl.BlockSpec((1, 1, nb, f), lambda i, b, s: (b, s, i, 0)),
            pl.BlockSpec((nb, f), lambda i, b, s: (i, 0)),
        ],
        out_specs=pl.BlockSpec((1, 1, nb, f), lambda i, b, s: (b, s, i, 0)),
        out_shape=jax.ShapeDtypeStruct((batch, t, n, f), x.dtype),
    )(xt, W)
    return jnp.transpose(out_t, (0, 2, 1, 3))


# speedup vs baseline: 6.1768x; 1.0106x over previous
import functools

import jax
import jax.numpy as jnp
from jax.experimental import pallas as pl


def _add_kernel(x_ref, w_ref, o_ref):
    o_ref[...] = x_ref[...] + w_ref[...][None, None, :, :]


def kernel(x, W):
    batch, n, t, f = x.shape
    xt = jnp.transpose(x, (0, 2, 1, 3))
    nb = 10000
    ts = 2
    out_t = pl.pallas_call(
        _add_kernel,
        grid=(n // nb, batch, t // ts),
        in_specs=[
            pl.BlockSpec((1, ts, nb, f), lambda i, b, s: (b, s, i, 0)),
            pl.BlockSpec((nb, f), lambda i, b, s: (i, 0)),
        ],
        out_specs=pl.BlockSpec((1, ts, nb, f), lambda i, b, s: (b, s, i, 0)),
        out_shape=jax.ShapeDtypeStruct((batch, t, n, f), x.dtype),
    )(xt, W)
    return jnp.transpose(out_t, (0, 2, 1, 3))
